# baseline (speedup 1.0000x reference)
import functools

import jax
import jax.numpy as jnp
from jax import lax
from jax.experimental import pallas as pl
from jax.experimental.pallas import tpu as pltpu
from jax.experimental.pallas import tpu_sc as plsc

B, C, N, K = 4, 64, 10000, 16
NK = N * K
BH = B // 2
NSC = 32
IW = 128
RPC = 5
NROWS = BH * NK // IW
NCHUNKS = NROWS // RPC
CHUNK = RPC * IW

_sc_mesh = plsc.VectorSubcoreMesh(core_axis_name="c", subcore_axis_name="s")


@functools.partial(
    pl.kernel,
    mesh=_sc_mesh,
    compiler_params=pltpu.CompilerParams(
        use_tc_tiling_on_sc=False, needs_layout_passes=False
    ),
    out_type=jax.ShapeDtypeStruct((BH * NK, 2 * C), jnp.float32),
    scratch_types=[
        pltpu.VMEM((RPC, IW), jnp.int32),
        pltpu.VMEM((CHUNK, 2 * C), jnp.float32),
        pltpu.SemaphoreType.DMA,
    ],
)
def _sc_gather(ftab_hbm, idx_hbm, gath_hbm, idx_buf, rows_buf, sem):
    wid = lax.axis_index("s") * 2 + lax.axis_index("c")

    def step(t, carry):
        chunk_id = wid + NSC * t

        @pl.when(chunk_id < NCHUNKS)
        def _():
            r0 = chunk_id * RPC
            pltpu.sync_copy(idx_hbm.at[pl.ds(r0, RPC), :], idx_buf)
            cps = [
                pltpu.async_copy(
                    ftab_hbm.at[idx_buf.at[r]],
                    rows_buf.at[pl.ds(r * IW, IW), :],
                    sem,
                )
                for r in range(RPC)
            ]
            for cp in cps:
                cp.wait()
            pltpu.sync_copy(
                rows_buf, gath_hbm.at[pl.ds(chunk_id * CHUNK, CHUNK), :]
            )

        return carry

    lax.fori_loop(0, (NCHUNKS + NSC - 1) // NSC, step, 0)


_JB = 8000


def _concat_body(rpe_ref, gath_ref, out_alias_ref, out_ref):
    del out_alias_ref
    out_ref[0] = jnp.concatenate([rpe_ref[0], gath_ref[0][:, 0:C]], axis=1)


def _concat_body_first(rpe_ref, gath_ref, out_ref):
    out_ref[0] = jnp.concatenate([rpe_ref[0], gath_ref[0][:, 0:C]], axis=1)


def _tc_concat_half(h, rpe_h, gath_h, prev):
    out_shape = jax.ShapeDtypeStruct((B, NK, 2 * C), jnp.float32)
    in_specs = [
        pl.BlockSpec((1, _JB, C), lambda b, j: (b, j, 0)),
        pl.BlockSpec((1, _JB, 2 * C), lambda b, j: (b, j, 0)),
    ]
    out_spec = pl.BlockSpec((1, _JB, 2 * C), lambda b, j: (h * BH + b, j, 0))
    if prev is None:
        return pl.pallas_call(
            _concat_body_first,
            grid=(BH, NK // _JB),
            in_specs=in_specs,
            out_specs=out_spec,
            out_shape=out_shape,
        )(rpe_h, gath_h)
    return pl.pallas_call(
        _concat_body,
        grid=(BH, NK // _JB),
        in_specs=in_specs + [pl.BlockSpec(memory_space=pl.ANY)],
        out_specs=out_spec,
        out_shape=out_shape,
        input_output_aliases={2: 0},
    )(rpe_h, gath_h, prev)


def kernel(relative_position_encoding, features, neighbors):
    featT = jnp.transpose(features[:, :, :, 0], (0, 2, 1))
    ftab = jnp.pad(featT.reshape(B * N, C), ((0, 0), (0, C)))
    rpe_t = jnp.transpose(relative_position_encoding, (0, 2, 3, 1)).reshape(
        B, NK, C
    )
    idxg = neighbors + (jnp.arange(B, dtype=jnp.int32) * N)[:, None, None]
    out = None
    for h in range(2):
        idx_h = idxg[h * BH : (h + 1) * BH].reshape(NROWS, IW)
        gath_h = _sc_gather(ftab, idx_h).reshape(BH, NK, 2 * C)
        rpe_h = rpe_t[h * BH : (h + 1) * BH]
        out = _tc_concat_half(h, rpe_h, gath_h, out)
    return jnp.transpose(out.reshape(B, N, K, 2 * C), (0, 3, 1, 2))


# speedup vs baseline: 1.2552x; 1.2552x over previous
import functools

import jax
import jax.numpy as jnp
from jax import lax
from jax.experimental import pallas as pl
from jax.experimental.pallas import tpu as pltpu
from jax.experimental.pallas import tpu_sc as plsc

B, C, N, K = 4, 64, 10000, 16
NK = N * K
NSC = 32
IW = 128
RPC = 2
NROWS = B * NK // IW
NCHUNKS = NROWS // RPC
CHUNK = RPC * IW

_sc_mesh = plsc.VectorSubcoreMesh(core_axis_name="c", subcore_axis_name="s")


@functools.partial(
    pl.kernel,
    mesh=_sc_mesh,
    compiler_params=pltpu.CompilerParams(
        use_tc_tiling_on_sc=False, needs_layout_passes=False
    ),
    out_type=jax.ShapeDtypeStruct((B * NK, 2 * C), jnp.float32),
    scratch_types=[
        pltpu.VMEM((RPC, IW), jnp.int32),
        pltpu.VMEM((RPC, IW), jnp.int32),
        pltpu.VMEM((CHUNK, 2 * C), jnp.float32),
        pltpu.VMEM((CHUNK, 2 * C), jnp.float32),
        pltpu.SemaphoreType.DMA,
        pltpu.SemaphoreType.DMA,
        pltpu.SemaphoreType.DMA,
        pltpu.SemaphoreType.DMA,
    ],
)
def _sc_gather(
    ftab_hbm, idx_hbm, gath_hbm,
    idx_buf0, idx_buf1, rows_buf0, rows_buf1, sem_g0, sem_g1, sem_w0, sem_w1,
):
    wid = lax.axis_index("s") * 2 + lax.axis_index("c")
    idx_bufs = (idx_buf0, idx_buf1)
    rows_bufs = (rows_buf0, rows_buf1)
    sems_g = (sem_g0, sem_g1)
    sems_w = (sem_w0, sem_w1)
    niter = (NCHUNKS + 2 * NSC - 1) // (2 * NSC)

    def chunk_of(u, p):
        return wid + NSC * (2 * u + p)

    def wait_writeout(c, p):
        pltpu.make_async_copy(
            rows_bufs[p], gath_hbm.at[pl.ds(c * CHUNK, CHUNK), :], sems_w[p]
        ).wait()

    def step(u, carry):
        for p in range(2):
            cprev = chunk_of(u - 1, p)

            @pl.when(jnp.logical_and(u > 0, cprev < NCHUNKS))
            def _(cprev=cprev, p=p):
                wait_writeout(cprev, p)

        for p in range(2):
            c = chunk_of(u, p)

            @pl.when(c < NCHUNKS)
            def _(c=c, p=p):
                pltpu.sync_copy(
                    idx_hbm.at[pl.ds(c * RPC, RPC), :], idx_bufs[p]
                )
                for r in range(RPC):
                    pltpu.async_copy(
                        ftab_hbm.at[idx_bufs[p].at[r]],
                        rows_bufs[p].at[pl.ds(r * IW, IW), :],
                        sems_g[p],
                    )

        for p in range(2):
            c = chunk_of(u, p)

            @pl.when(c < NCHUNKS)
            def _(c=c, p=p):
                for r in range(RPC):
                    pltpu.make_async_copy(
                        ftab_hbm.at[idx_bufs[p].at[r]],
                        rows_bufs[p].at[pl.ds(r * IW, IW), :],
                        sems_g[p],
                    ).wait()
                pltpu.async_copy(
                    rows_bufs[p],
                    gath_hbm.at[pl.ds(c * CHUNK, CHUNK), :],
                    sems_w[p],
                )

        return carry

    lax.fori_loop(0, niter, step, 0)
    for p in range(2):
        c = chunk_of(niter - 1, p)

        @pl.when(c < NCHUNKS)
        def _(c=c, p=p):
            wait_writeout(c, p)


_JB = 8000


def _concat_body(rpe_ref, gath_ref, out_ref):
    out_ref[0] = jnp.concatenate(
        [rpe_ref[0], gath_ref[0][:, 0:C]], axis=1
    )


def _tc_concat(rpe_t, gath2):
    return pl.pallas_call(
        _concat_body,
        grid=(B, NK // _JB),
        in_specs=[
            pl.BlockSpec((1, _JB, C), lambda b, j: (b, j, 0)),
            pl.BlockSpec((1, _JB, 2 * C), lambda b, j: (b, j, 0)),
        ],
        out_specs=pl.BlockSpec((1, _JB, 2 * C), lambda b, j: (b, j, 0)),
        out_shape=jax.ShapeDtypeStruct((B, NK, 2 * C), jnp.float32),
    )(rpe_t, gath2)


def kernel(relative_position_encoding, features, neighbors):
    ftab = jnp.transpose(features[:, :, :, 0], (0, 2, 1)).reshape(B * N, C)
    ftab = jnp.pad(ftab, ((0, 0), (0, C)))
    rpe_t = jnp.transpose(relative_position_encoding, (0, 2, 3, 1)).reshape(
        B, NK, C
    )
    idxg = neighbors + (jnp.arange(B, dtype=jnp.int32) * N)[:, None, None]
    idxg = idxg.reshape(NROWS, IW)
    gath2 = _sc_gather(ftab, idxg).reshape(B, NK, 2 * C)
    out = _tc_concat(rpe_t, gath2)
    return jnp.transpose(out.reshape(B, N, K, 2 * C), (0, 3, 1, 2))
